# baseline (speedup 1.0000x reference)
import functools

import jax
import jax.numpy as jnp
from jax import lax
from jax.experimental import pallas as pl
from jax.experimental.pallas import tpu as pltpu
from jax.experimental.pallas import tpu_sc as plsc

H = 1024
W = 1024
K = 4
D = 32
L = 16
NC = 2
NS = 16
NCHUNK = 128
NBLK = 4


def _splat(val, dtype=jnp.int32):
    return jnp.full((L,), val, dtype)


def kernel(position, positions, embeddings, harmonics, neighbor_map):
    B = position.shape[0]
    NP = positions.shape[0]
    NW = NC * NS
    BW = B // NW
    R = BW * K
    G_Q = BW // L
    RB = R // NBLK
    CPB = RB // NCHUNK
    GPB = RB // L
    QGPB = BW // NBLK // L

    pyx = position.transpose(1, 0).reshape(2 * B)
    ptab1 = positions.transpose(1, 0).reshape(2 * NP)
    nmap1 = neighbor_map.reshape(H, W // 128, 128, K).transpose(
        0, 1, 3, 2).reshape(H * W * K)
    emb_p = lax.bitcast_convert_type(
        embeddings.astype(jnp.bfloat16).reshape(NP, D // 2, 2), jnp.int32)

    mesh = plsc.VectorSubcoreMesh(
        core_axis_name="c", subcore_axis_name="s", num_cores=NC,
        num_subcores=NS)

    @functools.partial(
        pl.kernel,
        out_type=jax.ShapeDtypeStruct((B, D), jnp.float32),
        mesh=mesh,
        compiler_params=pltpu.CompilerParams(
            needs_layout_passes=False, use_tc_tiling_on_sc=False),
        scratch_types=[
            pltpu.VMEM((BW,), jnp.float32),
            pltpu.VMEM((BW,), jnp.float32),
            pltpu.VMEM((BW,), jnp.float32),
            pltpu.VMEM((BW,), jnp.float32),
            pltpu.VMEM((BW,), jnp.int32),
            pltpu.VMEM((R,), jnp.int32),
            pltpu.VMEM((R,), jnp.int32),
            pltpu.VMEM((R,), jnp.int32),
            pltpu.VMEM((R,), jnp.float32),
            pltpu.VMEM((R,), jnp.float32),
            pltpu.VMEM((R, D // 2), jnp.int32),
            pltpu.VMEM((R,), jnp.float32),
            pltpu.VMEM((D,), jnp.float32),
            pltpu.VMEM((BW, D), jnp.float32),
            [pltpu.SemaphoreType.DMA] * NBLK,
            [pltpu.SemaphoreType.DMA] * NBLK,
            [pltpu.SemaphoreType.DMA] * NBLK,
        ],
    )
    def run(pyx_h, nmap_h, ptab_h, emb_h, harm_h, out_h,
            py_v, px_v, iyf_v, ixf_v, flat_v, qidx_v, eidx_v,
            xidx_v, yt_v, xt_v, erow_v, d_v, harm_v, out_v,
            sem_n, sem_p, sem_e):
        wid = lax.axis_index("s") * NC + lax.axis_index("c")
        base = wid * BW
        pltpu.sync_copy(pyx_h.at[pl.ds(base, BW)], py_v)
        pltpu.sync_copy(pyx_h.at[pl.ds(B + base, BW)], px_v)
        pltpu.sync_copy(harm_h, harm_v)

        iota = lax.iota(jnp.int32, L)

        def s1(g, carry):
            off = g * L
            py = py_v[pl.ds(off, L)]
            px = px_v[pl.ds(off, L)]
            iy = py.astype(jnp.int32)
            ix = px.astype(jnp.int32)
            iyf_v[pl.ds(off, L)] = iy.astype(jnp.float32)
            ixf_v[pl.ds(off, L)] = ix.astype(jnp.float32)
            flat_v[pl.ds(off, L)] = (
                iy * (W * K)
                + lax.shift_right_logical(ix, 7) * (128 * K)
                + jnp.bitwise_and(ix, 127))
            return carry

        lax.fori_loop(0, G_Q, s1, 0)

        def nbr_chunks(blk):
            for c in range(CPB):
                o = blk * RB + c * NCHUNK
                yield (nmap_h.at[qidx_v.at[pl.ds(o, NCHUNK)]],
                       eidx_v.at[pl.ds(o, NCHUNK)])

        def pos_chunks(blk):
            for c in range(CPB):
                o = blk * RB + c * NCHUNK
                yield (ptab_h.at[eidx_v.at[pl.ds(o, NCHUNK)]],
                       yt_v.at[pl.ds(o, NCHUNK)])
                yield (ptab_h.at[xidx_v.at[pl.ds(o, NCHUNK)]],
                       xt_v.at[pl.ds(o, NCHUNK)])

        def emb_chunks(blk):
            for c in range(CPB):
                o = blk * RB + c * NCHUNK
                yield (emb_h.at[eidx_v.at[pl.ds(o, NCHUNK)]],
                       erow_v.at[pl.ds(o, NCHUNK), :])

        def s2_grp(t, carry):
            off = t * L
            i = off + iota
            b = lax.shift_right_logical(i, 2)
            fl = plsc.load_gather(flat_v, [b])
            qidx_v[pl.ds(off, L)] = fl + lax.shift_left(
                jnp.bitwise_and(i, K - 1), 7)
            return carry

        for blk in range(NBLK):
            lax.fori_loop(blk * GPB, (blk + 1) * GPB, s2_grp, 0)
            for src, dst in nbr_chunks(blk):
                pltpu.async_copy(src, dst, sem_n[blk])

        def s3_grp(t, carry):
            off = t * L
            xidx_v[pl.ds(off, L)] = eidx_v[pl.ds(off, L)] + NP
            return carry

        for blk in range(NBLK):
            for src, dst in nbr_chunks(blk):
                pltpu.make_async_copy(src, dst, sem_n[blk]).wait()
            lax.fori_loop(blk * GPB, (blk + 1) * GPB, s3_grp, 0)
            for src, dst in pos_chunks(blk):
                pltpu.async_copy(src, dst, sem_p[blk])
            for src, dst in emb_chunks(blk):
                pltpu.async_copy(src, dst, sem_e[blk])

        def s4_grp(t, carry):
            off = t * L
            r = off + iota
            q = lax.shift_right_logical(r, 2)
            dy = yt_v[pl.ds(off, L)] - plsc.load_gather(iyf_v, [q])
            dx = xt_v[pl.ds(off, L)] - plsc.load_gather(ixf_v, [q])
            x2 = dy * dy + dx * dx
            ibits = plsc.bitcast(x2, jnp.int32)
            magic = _splat(0x5F3759DF) - lax.shift_right_logical(ibits, 1)
            rr = plsc.bitcast(magic, jnp.float32)
            half = x2 * 0.5
            rr = rr * (1.5 - half * rr * rr)
            rr = rr * (1.5 - half * rr * rr)
            rr = rr * (1.5 - half * rr * rr)
            d_v[pl.ds(off, L)] = x2 * rr
            return carry

        for blk in range(NBLK):
            for src, dst in pos_chunks(blk):
                pltpu.make_async_copy(src, dst, sem_p[blk]).wait()
            lax.fori_loop(blk * GPB, (blk + 1) * GPB, s4_grp, 0)

        hcs = [plsc.load_gather(harm_v, [_splat(c)]) for c in range(D)]

        def s5_grp(g, carry):
            b = g * L + iota
            r0 = b * K
            r1 = r0 + 1
            r2 = r0 + 2
            r3 = r0 + 3
            d0 = plsc.load_gather(d_v, [r0])
            d1 = plsc.load_gather(d_v, [r1])
            d2 = plsc.load_gather(d_v, [r2])
            d3 = plsc.load_gather(d_v, [r3])
            inv = 1.0 / (d0 + d1 + d2 + d3 + 1e-8)
            w0 = 1.0 - d0 * inv
            w1 = 1.0 - d1 * inv
            w2 = 1.0 - d2 * inv
            w3 = 1.0 - d3 * inv
            himask = _splat(-65536)
            for cp in range(D // 2):
                cc = _splat(cp)
                g0 = plsc.load_gather(erow_v, [r0, cc])
                g1 = plsc.load_gather(erow_v, [r1, cc])
                g2 = plsc.load_gather(erow_v, [r2, cc])
                g3 = plsc.load_gather(erow_v, [r3, cc])
                lo0 = plsc.bitcast(lax.shift_left(g0, 16), jnp.float32)
                lo1 = plsc.bitcast(lax.shift_left(g1, 16), jnp.float32)
                lo2 = plsc.bitcast(lax.shift_left(g2, 16), jnp.float32)
                lo3 = plsc.bitcast(lax.shift_left(g3, 16), jnp.float32)
                hi0 = plsc.bitcast(jnp.bitwise_and(g0, himask), jnp.float32)
                hi1 = plsc.bitcast(jnp.bitwise_and(g1, himask), jnp.float32)
                hi2 = plsc.bitcast(jnp.bitwise_and(g2, himask), jnp.float32)
                hi3 = plsc.bitcast(jnp.bitwise_and(g3, himask), jnp.float32)
                acc_e = (w0 * lo0 + w1 * lo1 + w2 * lo2
                         + w3 * lo3) * hcs[2 * cp]
                acc_o = (w0 * hi0 + w1 * hi1 + w2 * hi2
                         + w3 * hi3) * hcs[2 * cp + 1]
                plsc.store_scatter(out_v, [b, _splat(2 * cp)], acc_e)
                plsc.store_scatter(out_v, [b, _splat(2 * cp + 1)], acc_o)
            return carry

        for blk in range(NBLK):
            for src, dst in emb_chunks(blk):
                pltpu.make_async_copy(src, dst, sem_e[blk]).wait()
            lax.fori_loop(blk * QGPB, (blk + 1) * QGPB, s5_grp, 0)

        pltpu.sync_copy(out_v, out_h.at[pl.ds(base, BW), :])

    return run(pyx, nmap1, ptab1, emb_p, harmonics)


# speedup vs baseline: 1.0782x; 1.0782x over previous
import functools

import jax
import jax.numpy as jnp
from jax import lax
from jax.experimental import pallas as pl
from jax.experimental.pallas import tpu as pltpu
from jax.experimental.pallas import tpu_sc as plsc

H = 1024
W = 1024
K = 4
D = 32
L = 16
NC = 2
NS = 16
NCHUNK = 128
NBLK = 4


def _splat(val, dtype=jnp.int32):
    return jnp.full((L,), val, dtype)


def kernel(position, positions, embeddings, harmonics, neighbor_map):
    B = position.shape[0]
    NP = positions.shape[0]
    NW = NC * NS
    BW = B // NW
    R = BW * K
    G_Q = BW // L
    RB = R // NBLK
    CPB = RB // NCHUNK
    GPB = RB // L
    QGPB = BW // NBLK // L

    pyx = position.transpose(1, 0).reshape(2 * B)
    ptab8 = positions.reshape(NP // 4, 8)
    nmap1 = neighbor_map.reshape(H, W // 128, 128, K).transpose(
        0, 1, 3, 2).reshape(H * W * K)
    emb_p = embeddings

    mesh = plsc.VectorSubcoreMesh(
        core_axis_name="c", subcore_axis_name="s", num_cores=NC,
        num_subcores=NS)

    @functools.partial(
        pl.kernel,
        out_type=jax.ShapeDtypeStruct((B, D), jnp.float32),
        mesh=mesh,
        compiler_params=pltpu.CompilerParams(
            needs_layout_passes=False, use_tc_tiling_on_sc=False),
        scratch_types=[
            pltpu.VMEM((BW,), jnp.float32),
            pltpu.VMEM((BW,), jnp.float32),
            pltpu.VMEM((BW,), jnp.float32),
            pltpu.VMEM((BW,), jnp.float32),
            pltpu.VMEM((BW,), jnp.int32),
            pltpu.VMEM((R,), jnp.int32),
            pltpu.VMEM((R,), jnp.int32),
            pltpu.VMEM((R,), jnp.int32),
            pltpu.VMEM((R, 8), jnp.float32),
            pltpu.VMEM((R, D), jnp.float32),
            pltpu.VMEM((R,), jnp.float32),
            pltpu.VMEM((D,), jnp.float32),
            pltpu.VMEM((BW, D), jnp.float32),
            [pltpu.SemaphoreType.DMA] * NBLK,
            [pltpu.SemaphoreType.DMA] * NBLK,
            [pltpu.SemaphoreType.DMA] * NBLK,
        ],
    )
    def run(pyx_h, nmap_h, ptab_h, emb_h, harm_h, out_h,
            py_v, px_v, iyf_v, ixf_v, flat_v, qidx_v, eidx_v,
            pidx_v, prow8_v, erow_v, d_v, harm_v, out_v,
            sem_n, sem_p, sem_e):
        wid = lax.axis_index("s") * NC + lax.axis_index("c")
        base = wid * BW
        pltpu.sync_copy(pyx_h.at[pl.ds(base, BW)], py_v)
        pltpu.sync_copy(pyx_h.at[pl.ds(B + base, BW)], px_v)
        pltpu.sync_copy(harm_h, harm_v)

        iota = lax.iota(jnp.int32, L)

        def s1(g, carry):
            off = g * L
            py = py_v[pl.ds(off, L)]
            px = px_v[pl.ds(off, L)]
            iy = py.astype(jnp.int32)
            ix = px.astype(jnp.int32)
            iyf_v[pl.ds(off, L)] = iy.astype(jnp.float32)
            ixf_v[pl.ds(off, L)] = ix.astype(jnp.float32)
            flat_v[pl.ds(off, L)] = (
                iy * (W * K)
                + lax.shift_right_logical(ix, 7) * (128 * K)
                + jnp.bitwise_and(ix, 127))
            return carry

        lax.fori_loop(0, G_Q, s1, 0)

        def nbr_chunks(blk):
            for c in range(CPB):
                o = blk * RB + c * NCHUNK
                yield (nmap_h.at[qidx_v.at[pl.ds(o, NCHUNK)]],
                       eidx_v.at[pl.ds(o, NCHUNK)])

        def pos_chunks(blk):
            for c in range(CPB):
                o = blk * RB + c * NCHUNK
                yield (ptab_h.at[pidx_v.at[pl.ds(o, NCHUNK)]],
                       prow8_v.at[pl.ds(o, NCHUNK), :])

        def emb_chunks(blk):
            for c in range(CPB):
                o = blk * RB + c * NCHUNK
                yield (emb_h.at[eidx_v.at[pl.ds(o, NCHUNK)]],
                       erow_v.at[pl.ds(o, NCHUNK), :])

        def s2_grp(t, carry):
            off = t * L
            i = off + iota
            b = lax.shift_right_logical(i, 2)
            fl = plsc.load_gather(flat_v, [b])
            qidx_v[pl.ds(off, L)] = fl + lax.shift_left(
                jnp.bitwise_and(i, K - 1), 7)
            return carry

        for blk in range(NBLK):
            lax.fori_loop(blk * GPB, (blk + 1) * GPB, s2_grp, 0)
            for src, dst in nbr_chunks(blk):
                pltpu.async_copy(src, dst, sem_n[blk])

        def s3_grp(t, carry):
            off = t * L
            pidx_v[pl.ds(off, L)] = lax.shift_right_logical(
                eidx_v[pl.ds(off, L)], 2)
            return carry

        for blk in range(NBLK):
            for src, dst in nbr_chunks(blk):
                pltpu.make_async_copy(src, dst, sem_n[blk]).wait()
            lax.fori_loop(blk * GPB, (blk + 1) * GPB, s3_grp, 0)
            for src, dst in pos_chunks(blk):
                pltpu.async_copy(src, dst, sem_p[blk])
            for src, dst in emb_chunks(blk):
                pltpu.async_copy(src, dst, sem_e[blk])

        def s4_grp(t, carry):
            off = t * L
            r = off + iota
            q = lax.shift_right_logical(r, 2)
            coly = lax.shift_left(
                jnp.bitwise_and(eidx_v[pl.ds(off, L)], 3), 1)
            yt = plsc.load_gather(prow8_v, [r, coly])
            xt = plsc.load_gather(prow8_v, [r, coly + 1])
            dy = yt - plsc.load_gather(iyf_v, [q])
            dx = xt - plsc.load_gather(ixf_v, [q])
            x2 = dy * dy + dx * dx
            ibits = plsc.bitcast(x2, jnp.int32)
            magic = _splat(0x5F3759DF) - lax.shift_right_logical(ibits, 1)
            rr = plsc.bitcast(magic, jnp.float32)
            half = x2 * 0.5
            rr = rr * (1.5 - half * rr * rr)
            rr = rr * (1.5 - half * rr * rr)
            rr = rr * (1.5 - half * rr * rr)
            d_v[pl.ds(off, L)] = x2 * rr
            return carry

        for blk in range(NBLK):
            for src, dst in pos_chunks(blk):
                pltpu.make_async_copy(src, dst, sem_p[blk]).wait()
            lax.fori_loop(blk * GPB, (blk + 1) * GPB, s4_grp, 0)

        hcs = [plsc.load_gather(harm_v, [_splat(c)]) for c in range(D)]

        def s5_grp(g, carry):
            b = g * L + iota
            r0 = b * K
            r1 = r0 + 1
            r2 = r0 + 2
            r3 = r0 + 3
            d0 = plsc.load_gather(d_v, [r0])
            d1 = plsc.load_gather(d_v, [r1])
            d2 = plsc.load_gather(d_v, [r2])
            d3 = plsc.load_gather(d_v, [r3])
            inv = 1.0 / (d0 + d1 + d2 + d3 + 1e-8)
            w0 = 1.0 - d0 * inv
            w1 = 1.0 - d1 * inv
            w2 = 1.0 - d2 * inv
            w3 = 1.0 - d3 * inv
            for col in range(D):
                cc = _splat(col)
                e0 = plsc.load_gather(erow_v, [r0, cc])
                e1 = plsc.load_gather(erow_v, [r1, cc])
                e2 = plsc.load_gather(erow_v, [r2, cc])
                e3 = plsc.load_gather(erow_v, [r3, cc])
                acc = (w0 * e0 + w1 * e1 + w2 * e2 + w3 * e3) * hcs[col]
                plsc.store_scatter(out_v, [b, cc], acc)
            return carry

        for blk in range(NBLK):
            for src, dst in emb_chunks(blk):
                pltpu.make_async_copy(src, dst, sem_e[blk]).wait()
            lax.fori_loop(blk * QGPB, (blk + 1) * QGPB, s5_grp, 0)

        pltpu.sync_copy(out_v, out_h.at[pl.ds(base, BW), :])

    return run(pyx, nmap1, ptab8, emb_p, harmonics)


# speedup vs baseline: 1.4373x; 1.3330x over previous
import functools

import jax
import jax.numpy as jnp
from jax import lax
from jax.experimental import pallas as pl
from jax.experimental.pallas import tpu as pltpu
from jax.experimental.pallas import tpu_sc as plsc

H = 1024
W = 1024
K = 4
D = 32
L = 16
NC = 2
NS = 16
NCHUNK = 128
NBLK = 4


def _splat(val, dtype=jnp.int32):
    return jnp.full((L,), val, dtype)


def kernel(position, positions, embeddings, harmonics, neighbor_map):
    B = position.shape[0]
    NP = positions.shape[0]
    NW = NC * NS
    BW = B // NW
    R = BW * K
    G_Q = BW // L
    RB = R // NBLK
    CPB = RB // NCHUNK
    GPB = RB // L
    QGPB = BW // NBLK // L

    pyx = position.transpose(1, 0).reshape(2 * B)
    ptab1 = positions.transpose(1, 0).reshape(2 * NP)
    nmap1 = neighbor_map.reshape(H, W // 128, 128, K).transpose(
        0, 1, 3, 2).reshape(H * W * K)
    emb_p = embeddings

    mesh = plsc.VectorSubcoreMesh(
        core_axis_name="c", subcore_axis_name="s", num_cores=NC,
        num_subcores=NS)

    @functools.partial(
        pl.kernel,
        out_type=jax.ShapeDtypeStruct((B, D), jnp.float32),
        mesh=mesh,
        compiler_params=pltpu.CompilerParams(
            needs_layout_passes=False, use_tc_tiling_on_sc=False),
        scratch_types=[
            pltpu.VMEM((BW,), jnp.float32),
            pltpu.VMEM((BW,), jnp.float32),
            pltpu.VMEM((BW,), jnp.float32),
            pltpu.VMEM((BW,), jnp.float32),
            pltpu.VMEM((BW,), jnp.int32),
            pltpu.VMEM((R,), jnp.int32),
            pltpu.VMEM((R,), jnp.int32),
            pltpu.VMEM((R,), jnp.int32),
            pltpu.VMEM((R,), jnp.float32),
            pltpu.VMEM((R,), jnp.float32),
            pltpu.VMEM((R, D), jnp.float32),
            pltpu.VMEM((R,), jnp.float32),
            pltpu.VMEM((D,), jnp.float32),
            pltpu.VMEM((BW, D), jnp.float32),
            [pltpu.SemaphoreType.DMA] * NBLK,
            [pltpu.SemaphoreType.DMA] * NBLK,
            [pltpu.SemaphoreType.DMA] * NBLK,
        ],
    )
    def run(pyx_h, nmap_h, ptab_h, emb_h, harm_h, out_h,
            py_v, px_v, iyf_v, ixf_v, flat_v, qidx_v, eidx_v,
            xidx_v, yt_v, xt_v, erow_v, d_v, harm_v, out_v,
            sem_n, sem_p, sem_e):
        wid = lax.axis_index("s") * NC + lax.axis_index("c")
        base = wid * BW
        pltpu.sync_copy(pyx_h.at[pl.ds(base, BW)], py_v)
        pltpu.sync_copy(pyx_h.at[pl.ds(B + base, BW)], px_v)
        pltpu.sync_copy(harm_h, harm_v)

        iota = lax.iota(jnp.int32, L)

        def s1(g, carry):
            off = g * L
            py = py_v[pl.ds(off, L)]
            px = px_v[pl.ds(off, L)]
            iy = py.astype(jnp.int32)
            ix = px.astype(jnp.int32)
            iyf_v[pl.ds(off, L)] = iy.astype(jnp.float32)
            ixf_v[pl.ds(off, L)] = ix.astype(jnp.float32)
            flat_v[pl.ds(off, L)] = (
                iy * (W * K)
                + lax.shift_right_logical(ix, 7) * (128 * K)
                + jnp.bitwise_and(ix, 127))
            return carry

        lax.fori_loop(0, G_Q, s1, 0)

        def nbr_chunks(blk):
            for c in range(CPB):
                o = blk * RB + c * NCHUNK
                yield (nmap_h.at[qidx_v.at[pl.ds(o, NCHUNK)]],
                       eidx_v.at[pl.ds(o, NCHUNK)])

        def pos_chunks(blk):
            for c in range(CPB):
                o = blk * RB + c * NCHUNK
                yield (ptab_h.at[eidx_v.at[pl.ds(o, NCHUNK)]],
                       yt_v.at[pl.ds(o, NCHUNK)])
                yield (ptab_h.at[xidx_v.at[pl.ds(o, NCHUNK)]],
                       xt_v.at[pl.ds(o, NCHUNK)])

        def emb_chunks(blk):
            for c in range(CPB):
                o = blk * RB + c * NCHUNK
                yield (emb_h.at[eidx_v.at[pl.ds(o, NCHUNK)]],
                       erow_v.at[pl.ds(o, NCHUNK), :])

        def s2_grp(t, carry):
            off = t * L
            i = off + iota
            b = lax.shift_right_logical(i, 2)
            fl = plsc.load_gather(flat_v, [b])
            qidx_v[pl.ds(off, L)] = fl + lax.shift_left(
                jnp.bitwise_and(i, K - 1), 7)
            return carry

        for blk in range(NBLK):
            lax.fori_loop(blk * GPB, (blk + 1) * GPB, s2_grp, 0)
            for src, dst in nbr_chunks(blk):
                pltpu.async_copy(src, dst, sem_n[blk])

        def s3_grp(t, carry):
            off = t * L
            xidx_v[pl.ds(off, L)] = eidx_v[pl.ds(off, L)] + NP
            return carry

        for blk in range(NBLK):
            for src, dst in nbr_chunks(blk):
                pltpu.make_async_copy(src, dst, sem_n[blk]).wait()
            lax.fori_loop(blk * GPB, (blk + 1) * GPB, s3_grp, 0)
            for src, dst in pos_chunks(blk):
                pltpu.async_copy(src, dst, sem_p[blk])
            for src, dst in emb_chunks(blk):
                pltpu.async_copy(src, dst, sem_e[blk])

        def s4_grp(t, carry):
            off = t * L
            r = off + iota
            q = lax.shift_right_logical(r, 2)
            dy = yt_v[pl.ds(off, L)] - plsc.load_gather(iyf_v, [q])
            dx = xt_v[pl.ds(off, L)] - plsc.load_gather(ixf_v, [q])
            x2 = dy * dy + dx * dx
            ibits = plsc.bitcast(x2, jnp.int32)
            magic = _splat(0x5F3759DF) - lax.shift_right_logical(ibits, 1)
            rr = plsc.bitcast(magic, jnp.float32)
            half = x2 * 0.5
            rr = rr * (1.5 - half * rr * rr)
            rr = rr * (1.5 - half * rr * rr)
            rr = rr * (1.5 - half * rr * rr)
            d_v[pl.ds(off, L)] = x2 * rr
            return carry

        for blk in range(NBLK):
            for src, dst in pos_chunks(blk):
                pltpu.make_async_copy(src, dst, sem_p[blk]).wait()
            lax.fori_loop(blk * GPB, (blk + 1) * GPB, s4_grp, 0)

        hcs = [plsc.load_gather(harm_v, [_splat(c)]) for c in range(D)]

        def s5_grp(g, carry):
            b = g * L + iota
            r0 = b * K
            r1 = r0 + 1
            r2 = r0 + 2
            r3 = r0 + 3
            d0 = plsc.load_gather(d_v, [r0])
            d1 = plsc.load_gather(d_v, [r1])
            d2 = plsc.load_gather(d_v, [r2])
            d3 = plsc.load_gather(d_v, [r3])
            inv = 1.0 / (d0 + d1 + d2 + d3 + 1e-8)
            w0 = 1.0 - d0 * inv
            w1 = 1.0 - d1 * inv
            w2 = 1.0 - d2 * inv
            w3 = 1.0 - d3 * inv
            for col in range(D):
                cc = _splat(col)
                e0 = plsc.load_gather(erow_v, [r0, cc])
                e1 = plsc.load_gather(erow_v, [r1, cc])
                e2 = plsc.load_gather(erow_v, [r2, cc])
                e3 = plsc.load_gather(erow_v, [r3, cc])
                acc = (w0 * e0 + w1 * e1 + w2 * e2 + w3 * e3) * hcs[col]
                plsc.store_scatter(out_v, [b, cc], acc)
            return carry

        for blk in range(NBLK):
            for src, dst in emb_chunks(blk):
                pltpu.make_async_copy(src, dst, sem_e[blk]).wait()
            lax.fori_loop(blk * QGPB, (blk + 1) * QGPB, s5_grp, 0)

        pltpu.sync_copy(out_v, out_h.at[pl.ds(base, BW), :])

    return run(pyx, nmap1, ptab1, emb_p, harmonics)
